# baseline (speedup 1.0000x reference)
import functools
import math

import jax
import jax.numpy as jnp
from jax import lax
from jax.experimental import pallas as pl
from jax.experimental.pallas import tpu as pltpu
from jax.experimental.pallas import tpu_sc as plsc

B, S, DIM, HEADS = 1, 2048, 1024, 16
HEAD = DIM // HEADS
TEN, TOPK, EDIM, SDIM = 8, 2, 512, 1024
EPS, THETA, RSF = 1e-5, 10000.0, 1.0

SB = 256
QB = 512
NEG = -1e30

f32 = jnp.float32
bf16 = jnp.bfloat16


KC = 256
HALF = HEAD // 2


def _k0_body(x_ref, w_ref, xn_ref, ta_ref, tb_ref):
    i = pl.program_id(0)
    x = x_ref[...]
    xn = x * jax.lax.rsqrt(jnp.mean(x * x, axis=-1, keepdims=True) + EPS)
    xn_ref[...] = (xn * w_ref[...]).astype(bf16)
    pos = jax.lax.broadcasted_iota(jnp.int32, (SB, KC), 0).astype(f32) + i * SB
    colv = jax.lax.broadcasted_iota(jnp.int32, (SB, KC), 1)
    k = (colv % HALF).astype(f32)
    inv = jnp.exp(k * (2.0 / HEAD) * math.log(1.0 / THETA))
    fr = pos * inv
    ta_ref[...] = jnp.cos(fr)
    sgn = jnp.where((colv // HALF) % 2 == 0, 1.0, -1.0)
    tb_ref[...] = jnp.sin(fr) * sgn


def _k0(x, attn_norm_w):
    return pl.pallas_call(
        _k0_body,
        grid=(S // SB,),
        in_specs=[
            pl.BlockSpec((SB, DIM), lambda i: (i, 0)),
            pl.BlockSpec((1, DIM), lambda i: (0, 0)),
        ],
        out_specs=[
            pl.BlockSpec((SB, DIM), lambda i: (i, 0)),
            pl.BlockSpec((SB, KC), lambda i: (i, 0)),
            pl.BlockSpec((SB, KC), lambda i: (i, 0)),
        ],
        out_shape=[
            jax.ShapeDtypeStruct((S, DIM), bf16),
            jax.ShapeDtypeStruct((S, KC), f32),
            jax.ShapeDtypeStruct((S, KC), f32),
        ],
    )(x, attn_norm_w)


NH_BLK = KC // HEAD


def _k1_prep(r, ta, tb, scale):
    col = jax.lax.broadcasted_iota(jnp.int32, (KC, KC), 0)
    row = jax.lax.broadcasted_iota(jnp.int32, (KC, KC), 1)
    gmat = (col // HEAD == row // HEAD).astype(f32)
    pmat = (row == (col // HEAD) * HEAD + (col % HEAD + HALF) % HEAD).astype(f32)
    z = r * r
    ss = jax.lax.dot_general(z, gmat, (((1,), (0,)), ((), ())),
                             preferred_element_type=f32)
    yn = r / jnp.maximum(jnp.sqrt(ss), EPS)
    sw = jax.lax.dot_general(yn, pmat, (((1,), (0,)), ((), ())),
                             preferred_element_type=f32)
    return ((yn * ta + sw * tb) * scale).astype(bf16)


def _k1_body(xn_ref, w_ref, ta_ref, tb_ref, out_ref):
    c = pl.program_id(1)
    r = jnp.dot(xn_ref[...], w_ref[...].astype(bf16),
                preferred_element_type=f32)
    nq = DIM // KC
    scale = 1.0 / math.sqrt(float(HEAD))

    @pl.when(c < nq)
    def _():
        out_ref[...] = _k1_prep(r, ta_ref[...], tb_ref[...], scale)

    @pl.when((c >= nq) & (c < 2 * nq))
    def _():
        out_ref[...] = _k1_prep(r, ta_ref[...], tb_ref[...], 1.0)

    @pl.when(c >= 2 * nq)
    def _():
        out_ref[...] = r.astype(bf16)


def _k1(xn, qkv_w, ta, tb):
    NC = 3 * DIM // KC
    return pl.pallas_call(
        _k1_body,
        grid=(S // QB, NC),
        in_specs=[
            pl.BlockSpec((QB, DIM), lambda s, c: (s, 0)),
            pl.BlockSpec((DIM, KC), lambda s, c: (0, c)),
            pl.BlockSpec((QB, KC), lambda s, c: (s, 0)),
            pl.BlockSpec((QB, KC), lambda s, c: (s, 0)),
        ],
        out_specs=pl.BlockSpec((QB, KC), lambda s, c: (s, c)),
        out_shape=jax.ShapeDtypeStruct((S, 3 * DIM), bf16),
    )(xn, qkv_w, ta, tb)


def _attn_one_head(q, k_ref, v_ref, sl, qi):

    def chunk(kj, masked):
        kc = k_ref[pl.ds(kj * QB, QB), sl]
        vc = v_ref[pl.ds(kj * QB, QB), sl]
        l = jax.lax.dot_general(q, kc, (((1,), (1,)), ((), ())),
                                preferred_element_type=f32).astype(bf16)
        one = jnp.array(1.0, bf16)
        p = one + l * (one + l * (jnp.array(0.5, bf16)
                                  + l * jnp.array(1.0 / 6.0, bf16)))
        if masked:
            row = jax.lax.broadcasted_iota(jnp.int32, (QB, QB), 0)
            col = jax.lax.broadcasted_iota(jnp.int32, (QB, QB), 1)
            p = jnp.where(col <= row, p, jnp.array(0.0, bf16))
        o = jnp.dot(p, vc, preferred_element_type=f32)
        return o, jnp.sum(p, axis=-1, keepdims=True).astype(f32)

    def body(kj, carry):
        o_acc, s_acc = carry
        o, s = chunk(kj, masked=False)
        return o_acc + o, s_acc + s

    o_acc, s_acc = jax.lax.fori_loop(
        0, qi, body,
        (jnp.zeros((QB, HEAD), f32), jnp.zeros((QB, 1), f32)))
    o, s = chunk(qi, masked=True)
    return (o_acc + o) / (s_acc + s)


def _k2_body(q_ref, k_ref, v_ref, out_ref):
    qi = pl.program_id(1)
    outs = []
    for i in range(2):
        sl = slice(i * HEAD, (i + 1) * HEAD)
        outs.append(_attn_one_head(q_ref[:, sl], k_ref, v_ref, sl, qi))
    out_ref[...] = jnp.concatenate(outs, axis=-1).astype(bf16)


def _k2(qkv):
    HP = HEADS // 2
    return pl.pallas_call(
        _k2_body,
        grid=(HP, S // QB),
        in_specs=[
            pl.BlockSpec((QB, 2 * HEAD), lambda h, qi: (qi, h)),
            pl.BlockSpec((S, 2 * HEAD), lambda h, qi: (0, HP + h)),
            pl.BlockSpec((S, 2 * HEAD), lambda h, qi: (0, 2 * HP + h)),
        ],
        out_specs=pl.BlockSpec((QB, 2 * HEAD), lambda h, qi: (qi, h)),
        out_shape=jax.ShapeDtypeStruct((S, DIM), bf16),
    )(qkv, qkv, qkv)


def _k3_body(attn_ref, ow_ref, x_ref, fw_ref, keys_ref, idx_ref, val_ref,
             resid_ref, xffn_ref, scores_ref):
    att = jnp.dot(attn_ref[...], ow_ref[...], preferred_element_type=f32)
    resid = att + x_ref[...]
    resid_ref[...] = resid
    xn = resid * jax.lax.rsqrt(jnp.mean(resid * resid, axis=-1, keepdims=True) + EPS)
    xn = xn * fw_ref[...]
    xffn_ref[...] = xn.astype(bf16)
    tv = jnp.dot(xn, keys_ref[...], preferred_element_type=f32)
    idx = idx_ref[...]
    tvsel = jnp.zeros_like(tv)
    for e in range(TEN):
        tvsel = tvsel + tv[:, e:e + 1] * (idx == e).astype(f32)
    scores_ref[...] = jax.nn.sigmoid(val_ref[...] + tvsel) * RSF


def _k3(attn, o_wb, x_input, ffn_norm_w, keys_pad, idx_pad, val_pad):
    return pl.pallas_call(
        _k3_body,
        grid=(S // SB,),
        in_specs=[
            pl.BlockSpec((SB, DIM), lambda i: (i, 0)),
            pl.BlockSpec((DIM, DIM), lambda i: (0, 0)),
            pl.BlockSpec((SB, DIM), lambda i: (i, 0)),
            pl.BlockSpec((1, DIM), lambda i: (0, 0)),
            pl.BlockSpec((DIM, 128), lambda i: (0, 0)),
            pl.BlockSpec((SB, 128), lambda i: (i, 0)),
            pl.BlockSpec((SB, 128), lambda i: (i, 0)),
        ],
        out_specs=[
            pl.BlockSpec((SB, DIM), lambda i: (i, 0)),
            pl.BlockSpec((SB, DIM), lambda i: (i, 0)),
            pl.BlockSpec((SB, 128), lambda i: (i, 0)),
        ],
        out_shape=[
            jax.ShapeDtypeStruct((S, DIM), f32),
            jax.ShapeDtypeStruct((S, DIM), bf16),
            jax.ShapeDtypeStruct((S, 128), f32),
        ],
    )(attn, o_wb, x_input, ffn_norm_w, keys_pad, idx_pad, val_pad)


def _k4_body(x_ref, w0_ref, w1_ref, w2_ref, idx_ref, sc_ref, y_ref):
    e = pl.program_id(0)
    x = x_ref[...]
    g = jnp.dot(x, w0_ref[0].astype(bf16), preferred_element_type=f32)
    u = jnp.dot(x, w1_ref[0].astype(bf16), preferred_element_type=f32)
    h = (jax.nn.silu(g) * u).astype(bf16)
    o = jax.lax.dot_general(h, w2_ref[0].astype(bf16), (((1,), (1,)), ((), ())),
                            preferred_element_type=f32)
    w = jnp.sum(sc_ref[...] * (idx_ref[...] == e).astype(f32), axis=-1,
                keepdims=True)
    contrib = o * w

    @pl.when(e == 0)
    def _():
        y_ref[...] = contrib

    @pl.when(e > 0)
    def _():
        y_ref[...] = y_ref[...] + contrib


def _k4(xffn, w0, w1, w2, idx_pad, scores):
    return pl.pallas_call(
        _k4_body,
        grid=(TEN,),
        in_specs=[
            pl.BlockSpec((S, DIM), lambda e: (0, 0)),
            pl.BlockSpec((1, DIM, EDIM), lambda e: (e, 0, 0)),
            pl.BlockSpec((1, DIM, EDIM), lambda e: (e, 0, 0)),
            pl.BlockSpec((1, DIM, EDIM), lambda e: (e, 0, 0)),
            pl.BlockSpec((S, 128), lambda e: (0, 0)),
            pl.BlockSpec((S, 128), lambda e: (0, 0)),
        ],
        out_specs=pl.BlockSpec((S, DIM), lambda e: (0, 0)),
        out_shape=jax.ShapeDtypeStruct((S, DIM), f32),
    )(xffn, w0, w1, w2, idx_pad, scores)


T_ROW = 256
NT = (2 * S + TEN * T_ROW) // T_ROW
NP = NT * T_ROW
NW = 32
I32 = jnp.int32


def _k3b_body(i0_ref, i1_ref, pos0_ref, pos1_ref, teid_ref):
    i0, i1 = i0_ref[...], i1_ref[...]
    a = jax.lax.broadcasted_iota(I32, (128, 128), 0)
    b = jax.lax.broadcasted_iota(I32, (128, 128), 1)
    mlane = (a < b).astype(f32)
    ra = jax.lax.broadcasted_iota(I32, (16, 16), 0)
    rb = jax.lax.broadcasted_iota(I32, (16, 16), 1)
    mrow = (rb < ra).astype(f32)
    pos0 = jnp.zeros((16, 128), f32)
    pos1 = jnp.zeros((16, 128), f32)
    off = jnp.array(0.0, f32)
    offs = []
    for e in range(TEN):
        o0 = (i0 == e).astype(f32)
        o1 = (i1 == e).astype(f32)
        p0 = jax.lax.dot_general(o0, mlane, (((1,), (0,)), ((), ())),
                                 preferred_element_type=f32)
        p1 = jax.lax.dot_general(o1, mlane, (((1,), (0,)), ((), ())),
                                 preferred_element_type=f32)
        rt0 = jnp.sum(o0, axis=1, keepdims=True)
        rt1 = jnp.sum(o1, axis=1, keepdims=True)
        pre0 = jax.lax.dot_general(mrow, rt0, (((1,), (0,)), ((), ())),
                                   preferred_element_type=f32)
        pre1 = jax.lax.dot_general(mrow, rt1, (((1,), (0,)), ((), ())),
                                   preferred_element_type=f32)
        c0 = jnp.sum(rt0)
        cnt = c0 + jnp.sum(rt1)
        pos0 = pos0 + o0 * (p0 + pre0 + off)
        pos1 = pos1 + o1 * (p1 + pre1 + c0 + off)
        offs.append(off)
        off = off + jnp.ceil(cnt / T_ROW) * T_ROW
    pos0_ref[...] = pos0.astype(I32)
    pos1_ref[...] = pos1.astype(I32)
    m = jax.lax.broadcasted_iota(I32, (1, 128), 1).astype(f32) * T_ROW
    te = sum(((m >= o).astype(I32)) for o in offs) - 1
    teid_ref[...] = te


def _k3b(i0, i1):
    return pl.pallas_call(
        _k3b_body,
        grid=(1,),
        in_specs=[
            pl.BlockSpec((16, 128), lambda i: (0, 0)),
            pl.BlockSpec((16, 128), lambda i: (0, 0)),
        ],
        out_specs=[
            pl.BlockSpec((16, 128), lambda i: (0, 0)),
            pl.BlockSpec((16, 128), lambda i: (0, 0)),
            pl.BlockSpec((1, 128), lambda i: (0, 0)),
        ],
        out_shape=[
            jax.ShapeDtypeStruct((16, 128), I32),
            jax.ShapeDtypeStruct((16, 128), I32),
            jax.ShapeDtypeStruct((1, 128), I32),
        ],
    )(i0, i1)


@functools.lru_cache(maxsize=None)
def _sc_kernels():
    mesh = plsc.VectorSubcoreMesh(core_axis_name="c", subcore_axis_name="s")

    @functools.partial(
        pl.kernel,
        out_type=jax.ShapeDtypeStruct((NP, DIM2), I32),
        mesh=mesh,
        scratch_types=[
            pltpu.VMEM((_TPW,), I32),
            pltpu.VMEM((_TPW,), I32),
            pltpu.VMEM((_TPW, DIM2), I32),
            pltpu.SemaphoreType.DMA,
        ],
    )
    def _sc_dispatch(p0_hbm, p1_hbm, x_hbm, out_hbm, i0_v, i1_v, val_v, sem):
        wid = lax.axis_index("s") * 2 + lax.axis_index("c")
        base = wid * _TPW
        pltpu.sync_copy(p0_hbm.at[wid], i0_v)
        pltpu.sync_copy(p1_hbm.at[wid], i1_v)
        pltpu.sync_copy(x_hbm.at[pl.ds(base, _TPW)], val_v)
        pltpu.async_copy(val_v, out_hbm.at[i0_v], sem).wait()
        pltpu.async_copy(val_v, out_hbm.at[i1_v], sem).wait()

    @functools.partial(
        pl.kernel,
        out_type=[
            jax.ShapeDtypeStruct((S, DIM2), I32),
            jax.ShapeDtypeStruct((S, DIM2), I32),
        ],
        mesh=mesh,
        scratch_types=[
            pltpu.VMEM((_TPW,), I32),
            pltpu.VMEM((_TPW, DIM2), I32),
            pltpu.SemaphoreType.DMA,
        ],
    )
    def _sc_combine(p0_hbm, p1_hbm, o_hbm, y0_hbm, y1_hbm, i_v, rows_v, sem):
        wid = lax.axis_index("s") * 2 + lax.axis_index("c")
        base = wid * _TPW
        for src, dst in ((p0_hbm, y0_hbm), (p1_hbm, y1_hbm)):
            pltpu.sync_copy(src.at[wid], i_v)
            pltpu.async_copy(o_hbm.at[i_v], rows_v, sem).wait()
            pltpu.sync_copy(rows_v, dst.at[pl.ds(base, _TPW)])

    return _sc_dispatch, _sc_combine


_TPW = S // NW
DIM2 = DIM // 2


def _b2i(x):
    n, d = x.shape
    return jax.lax.bitcast_convert_type(x.reshape(n, d // 2, 2), I32)


def _i2b(x):
    n, d = x.shape
    return jax.lax.bitcast_convert_type(x, bf16).reshape(n, 2 * d)


def _gemm_body(teid_ref, xs_ref, w0_ref, w1_ref, w2_ref, o_ref):
    xs = xs_ref[...]
    g = jnp.dot(xs, w0_ref[0].astype(bf16), preferred_element_type=f32)
    u = jnp.dot(xs, w1_ref[0].astype(bf16), preferred_element_type=f32)
    h = (jax.nn.silu(g) * u).astype(bf16)
    o = jax.lax.dot_general(h, w2_ref[0].astype(bf16), (((1,), (1,)), ((), ())),
                            preferred_element_type=f32)
    o_ref[...] = o.astype(bf16)


def _gemm(teid, xs, w0, w1, w2):
    grid_spec = pltpu.PrefetchScalarGridSpec(
        num_scalar_prefetch=1,
        grid=(NT,),
        in_specs=[
            pl.BlockSpec((T_ROW, DIM), lambda m, t: (m, 0)),
            pl.BlockSpec((1, DIM, EDIM), lambda m, t: (t[m], 0, 0)),
            pl.BlockSpec((1, DIM, EDIM), lambda m, t: (t[m], 0, 0)),
            pl.BlockSpec((1, DIM, EDIM), lambda m, t: (t[m], 0, 0)),
        ],
        out_specs=pl.BlockSpec((T_ROW, DIM), lambda m, t: (m, 0)),
    )
    return pl.pallas_call(
        _gemm_body,
        grid_spec=grid_spec,
        out_shape=jax.ShapeDtypeStruct((NP, DIM), bf16),
    )(teid, xs, w0, w1, w2)


def _k5_body(x_ref, up_ref, down_ref, y0_ref, y1_ref, sc_ref, resid_ref,
             out_ref):
    xu = jnp.dot(x_ref[...], up_ref[...], preferred_element_type=f32)
    x1, x2 = xu[:, :SDIM], xu[:, SDIM:]
    h = (jax.nn.silu(x1) * x2).astype(bf16)
    ys = jnp.dot(h, down_ref[...], preferred_element_type=f32)
    s0 = sc_ref[:, 0:1]
    s1 = sc_ref[:, 1:2]
    y_moe = y0_ref[...].astype(f32) * s0 + y1_ref[...].astype(f32) * s1
    out_ref[...] = ys + y_moe + resid_ref[...]


def _k5(xffn, up_wb, down_wb, y0, y1, scores, resid):
    return pl.pallas_call(
        _k5_body,
        grid=(S // SB,),
        in_specs=[
            pl.BlockSpec((SB, DIM), lambda i: (i, 0)),
            pl.BlockSpec((DIM, 2 * SDIM), lambda i: (0, 0)),
            pl.BlockSpec((SDIM, DIM), lambda i: (0, 0)),
            pl.BlockSpec((SB, DIM), lambda i: (i, 0)),
            pl.BlockSpec((SB, DIM), lambda i: (i, 0)),
            pl.BlockSpec((SB, 128), lambda i: (i, 0)),
            pl.BlockSpec((SB, DIM), lambda i: (i, 0)),
        ],
        out_specs=pl.BlockSpec((SB, DIM), lambda i: (i, 0)),
        out_shape=jax.ShapeDtypeStruct((S, DIM), f32),
    )(xffn, up_wb, down_wb, y0, y1, scores, resid)


def kernel(x_input, indices, values, attn_norm_w, qkv_w, o_w, ffn_norm_w,
           keys_w, experts_w, up_w, down_w):
    x = x_input.reshape(S, DIM)
    o_wb = o_w.astype(bf16)
    up_wb = up_w.astype(bf16)
    down_wb = down_w.astype(bf16)
    keys_pad = jnp.pad(keys_w, ((0, 0), (0, 128 - TEN)))
    idx_pad = jnp.pad(indices.astype(jnp.int32), ((0, 0), (0, 128 - TOPK)),
                      constant_values=TEN)
    val_pad = jnp.pad(values, ((0, 0), (0, 128 - TOPK)), constant_values=NEG)

    xn, ta, tb = _k0(x, attn_norm_w.reshape(1, DIM))
    qkv = _k1(xn, qkv_w, ta, tb)
    attn = _k2(qkv)
    resid, xffn, scores = _k3(attn, o_wb, x, ffn_norm_w.reshape(1, DIM),
                              keys_pad, idx_pad, val_pad)
    i0 = indices[:, 0].astype(I32).reshape(16, 128)
    i1 = indices[:, 1].astype(I32).reshape(16, 128)
    pos0m, pos1m, teid_pad = _k3b(i0, i1)
    pos0 = pos0m.reshape(NW, _TPW)
    pos1 = pos1m.reshape(NW, _TPW)
    teid = teid_pad.reshape(128)[:NT]
    sc_dispatch, sc_combine = _sc_kernels()
    xs = _i2b(sc_dispatch(pos0, pos1, _b2i(xffn)))
    o_s = _gemm(teid, xs, experts_w[0], experts_w[1], experts_w[2])
    y0i, y1i = sc_combine(pos0, pos1, _b2i(o_s))
    y0, y1 = _i2b(y0i), _i2b(y1i)
    out = _k5(xffn, up_wb, down_wb, y0, y1, scores, resid)
    return out.reshape(B, S, DIM)


# speedup vs baseline: 2.0947x; 2.0947x over previous
import functools
import math

import jax
import jax.numpy as jnp
from jax import lax
from jax.experimental import pallas as pl
from jax.experimental.pallas import tpu as pltpu
from jax.experimental.pallas import tpu_sc as plsc

B, S, DIM, HEADS = 1, 2048, 1024, 16
HEAD = DIM // HEADS
TEN, TOPK, EDIM, SDIM = 8, 2, 512, 1024
EPS, THETA, RSF = 1e-5, 10000.0, 1.0

SB = 256
QB = 512
NEG = -1e30

f32 = jnp.float32
bf16 = jnp.bfloat16


KC = 256
HALF = HEAD // 2


def _k0_body(x_ref, w_ref, xn_ref, ta_ref, tb_ref):
    i = pl.program_id(0)
    x = x_ref[...]
    xn = x * jax.lax.rsqrt(jnp.mean(x * x, axis=-1, keepdims=True) + EPS)
    xn_ref[...] = (xn * w_ref[...]).astype(bf16)
    pos = jax.lax.broadcasted_iota(jnp.int32, (SB, KC), 0).astype(f32) + i * SB
    colv = jax.lax.broadcasted_iota(jnp.int32, (SB, KC), 1)
    k = (colv % HALF).astype(f32)
    inv = jnp.exp(k * (2.0 / HEAD) * math.log(1.0 / THETA))
    fr = pos * inv
    ta_ref[...] = jnp.cos(fr)
    sgn = jnp.where((colv // HALF) % 2 == 0, 1.0, -1.0)
    tb_ref[...] = jnp.sin(fr) * sgn


def _k0(x, attn_norm_w):
    return pl.pallas_call(
        _k0_body,
        grid=(S // SB,),
        in_specs=[
            pl.BlockSpec((SB, DIM), lambda i: (i, 0)),
            pl.BlockSpec((1, DIM), lambda i: (0, 0)),
        ],
        out_specs=[
            pl.BlockSpec((SB, DIM), lambda i: (i, 0)),
            pl.BlockSpec((SB, KC), lambda i: (i, 0)),
            pl.BlockSpec((SB, KC), lambda i: (i, 0)),
        ],
        out_shape=[
            jax.ShapeDtypeStruct((S, DIM), bf16),
            jax.ShapeDtypeStruct((S, KC), f32),
            jax.ShapeDtypeStruct((S, KC), f32),
        ],
    )(x, attn_norm_w)


NH_BLK = KC // HEAD


def _k1_prep(r, ta, tb, scale):
    col = jax.lax.broadcasted_iota(jnp.int32, (KC, KC), 0)
    row = jax.lax.broadcasted_iota(jnp.int32, (KC, KC), 1)
    gmat = (col // HEAD == row // HEAD).astype(f32)
    pmat = (row == (col // HEAD) * HEAD + (col % HEAD + HALF) % HEAD).astype(f32)
    z = r * r
    ss = jax.lax.dot_general(z, gmat, (((1,), (0,)), ((), ())),
                             preferred_element_type=f32)
    yn = r / jnp.maximum(jnp.sqrt(ss), EPS)
    sw = jax.lax.dot_general(yn, pmat, (((1,), (0,)), ((), ())),
                             preferred_element_type=f32)
    return ((yn * ta + sw * tb) * scale).astype(bf16)


def _k1_body(xn_ref, w_ref, ta_ref, tb_ref, out_ref):
    c = pl.program_id(1)
    r = jnp.dot(xn_ref[...], w_ref[...].astype(bf16),
                preferred_element_type=f32)
    nq = DIM // KC
    scale = 1.0 / math.sqrt(float(HEAD))

    @pl.when(c < nq)
    def _():
        out_ref[...] = _k1_prep(r, ta_ref[...], tb_ref[...], scale)

    @pl.when((c >= nq) & (c < 2 * nq))
    def _():
        out_ref[...] = _k1_prep(r, ta_ref[...], tb_ref[...], 1.0)

    @pl.when(c >= 2 * nq)
    def _():
        out_ref[...] = r.astype(bf16)


def _k1(xn, qkv_w, ta, tb):
    NC = 3 * DIM // KC
    return pl.pallas_call(
        _k1_body,
        grid=(S // QB, NC),
        in_specs=[
            pl.BlockSpec((QB, DIM), lambda s, c: (s, 0)),
            pl.BlockSpec((DIM, KC), lambda s, c: (0, c)),
            pl.BlockSpec((QB, KC), lambda s, c: (s, 0)),
            pl.BlockSpec((QB, KC), lambda s, c: (s, 0)),
        ],
        out_specs=pl.BlockSpec((QB, KC), lambda s, c: (s, c)),
        out_shape=jax.ShapeDtypeStruct((S, 3 * DIM), bf16),
    )(xn, qkv_w, ta, tb)


def _attn_one_head(q, k_ref, v_ref, sl, qi):

    def chunk(kj, masked):
        kc = k_ref[pl.ds(kj * QB, QB), sl]
        vc = v_ref[pl.ds(kj * QB, QB), sl]
        l = jax.lax.dot_general(q, kc, (((1,), (1,)), ((), ())),
                                preferred_element_type=f32).astype(bf16)
        one = jnp.array(1.0, bf16)
        p = one + l * (one + l * (jnp.array(0.5, bf16)
                                  + l * jnp.array(1.0 / 6.0, bf16)))
        if masked:
            row = jax.lax.broadcasted_iota(jnp.int32, (QB, QB), 0)
            col = jax.lax.broadcasted_iota(jnp.int32, (QB, QB), 1)
            p = jnp.where(col <= row, p, jnp.array(0.0, bf16))
        o = jnp.dot(p, vc, preferred_element_type=f32)
        return o, jnp.sum(p, axis=-1, keepdims=True).astype(f32)

    def body(kj, carry):
        o_acc, s_acc = carry
        o, s = chunk(kj, masked=False)
        return o_acc + o, s_acc + s

    o_acc, s_acc = jax.lax.fori_loop(
        0, qi, body,
        (jnp.zeros((QB, HEAD), f32), jnp.zeros((QB, 1), f32)))
    o, s = chunk(qi, masked=True)
    return (o_acc + o) / (s_acc + s)


def _k2_body(q_ref, k_ref, v_ref, out_ref):
    qi = pl.program_id(1)
    outs = []
    for i in range(2):
        sl = slice(i * HEAD, (i + 1) * HEAD)
        outs.append(_attn_one_head(q_ref[:, sl], k_ref, v_ref, sl, qi))
    out_ref[...] = jnp.concatenate(outs, axis=-1).astype(bf16)


def _k2(qkv):
    HP = HEADS // 2
    return pl.pallas_call(
        _k2_body,
        grid=(HP, S // QB),
        in_specs=[
            pl.BlockSpec((QB, 2 * HEAD), lambda h, qi: (qi, h)),
            pl.BlockSpec((S, 2 * HEAD), lambda h, qi: (0, HP + h)),
            pl.BlockSpec((S, 2 * HEAD), lambda h, qi: (0, 2 * HP + h)),
        ],
        out_specs=pl.BlockSpec((QB, 2 * HEAD), lambda h, qi: (qi, h)),
        out_shape=jax.ShapeDtypeStruct((S, DIM), bf16),
    )(qkv, qkv, qkv)


def _k3_body(attn_ref, ow_ref, x_ref, fw_ref, keys_ref, idx_ref, val_ref,
             resid_ref, xffn_ref, xffn32_ref, scores_ref):
    att = jnp.dot(attn_ref[...], ow_ref[...], preferred_element_type=f32)
    resid = att + x_ref[...]
    resid_ref[...] = resid
    xn = resid * jax.lax.rsqrt(jnp.mean(resid * resid, axis=-1, keepdims=True) + EPS)
    xn = xn * fw_ref[...]
    xffn_ref[...] = xn.astype(bf16)
    xffn32_ref[...] = xn
    tv = jnp.dot(xn, keys_ref[...], preferred_element_type=f32)
    idx = idx_ref[...]
    tvsel = jnp.zeros_like(tv)
    for e in range(TEN):
        tvsel = tvsel + tv[:, e:e + 1] * (idx == e).astype(f32)
    scores_ref[...] = jax.nn.sigmoid(val_ref[...] + tvsel) * RSF


def _k3(attn, o_wb, x_input, ffn_norm_w, keys_pad, idx_pad, val_pad):
    return pl.pallas_call(
        _k3_body,
        grid=(S // SB,),
        in_specs=[
            pl.BlockSpec((SB, DIM), lambda i: (i, 0)),
            pl.BlockSpec((DIM, DIM), lambda i: (0, 0)),
            pl.BlockSpec((SB, DIM), lambda i: (i, 0)),
            pl.BlockSpec((1, DIM), lambda i: (0, 0)),
            pl.BlockSpec((DIM, 128), lambda i: (0, 0)),
            pl.BlockSpec((SB, 128), lambda i: (i, 0)),
            pl.BlockSpec((SB, 128), lambda i: (i, 0)),
        ],
        out_specs=[
            pl.BlockSpec((SB, DIM), lambda i: (i, 0)),
            pl.BlockSpec((SB, DIM), lambda i: (i, 0)),
            pl.BlockSpec((SB, DIM), lambda i: (i, 0)),
            pl.BlockSpec((SB, 128), lambda i: (i, 0)),
        ],
        out_shape=[
            jax.ShapeDtypeStruct((S, DIM), f32),
            jax.ShapeDtypeStruct((S, DIM), bf16),
            jax.ShapeDtypeStruct((S, DIM), f32),
            jax.ShapeDtypeStruct((S, 128), f32),
        ],
    )(attn, o_wb, x_input, ffn_norm_w, keys_pad, idx_pad, val_pad)


def _k4_body(x_ref, w0_ref, w1_ref, w2_ref, idx_ref, sc_ref, y_ref):
    e = pl.program_id(0)
    x = x_ref[...]
    g = jnp.dot(x, w0_ref[0].astype(bf16), preferred_element_type=f32)
    u = jnp.dot(x, w1_ref[0].astype(bf16), preferred_element_type=f32)
    h = (jax.nn.silu(g) * u).astype(bf16)
    o = jax.lax.dot_general(h, w2_ref[0].astype(bf16), (((1,), (1,)), ((), ())),
                            preferred_element_type=f32)
    w = jnp.sum(sc_ref[...] * (idx_ref[...] == e).astype(f32), axis=-1,
                keepdims=True)
    contrib = o * w

    @pl.when(e == 0)
    def _():
        y_ref[...] = contrib

    @pl.when(e > 0)
    def _():
        y_ref[...] = y_ref[...] + contrib


def _k4(xffn, w0, w1, w2, idx_pad, scores):
    return pl.pallas_call(
        _k4_body,
        grid=(TEN,),
        in_specs=[
            pl.BlockSpec((S, DIM), lambda e: (0, 0)),
            pl.BlockSpec((1, DIM, EDIM), lambda e: (e, 0, 0)),
            pl.BlockSpec((1, DIM, EDIM), lambda e: (e, 0, 0)),
            pl.BlockSpec((1, DIM, EDIM), lambda e: (e, 0, 0)),
            pl.BlockSpec((S, 128), lambda e: (0, 0)),
            pl.BlockSpec((S, 128), lambda e: (0, 0)),
        ],
        out_specs=pl.BlockSpec((S, DIM), lambda e: (0, 0)),
        out_shape=jax.ShapeDtypeStruct((S, DIM), f32),
    )(xffn, w0, w1, w2, idx_pad, scores)


T_ROW = 256
NT = (2 * S + TEN * T_ROW) // T_ROW
NP = NT * T_ROW
NW = 32
I32 = jnp.int32


def _k3b_body(i0_ref, i1_ref, pos0_ref, pos1_ref, teid_ref):
    i0, i1 = i0_ref[...], i1_ref[...]
    a = jax.lax.broadcasted_iota(I32, (128, 128), 0)
    b = jax.lax.broadcasted_iota(I32, (128, 128), 1)
    mlane = (a < b).astype(f32)
    ra = jax.lax.broadcasted_iota(I32, (16, 16), 0)
    rb = jax.lax.broadcasted_iota(I32, (16, 16), 1)
    mrow = (rb < ra).astype(f32)
    pos0 = jnp.zeros((16, 128), f32)
    pos1 = jnp.zeros((16, 128), f32)
    off = jnp.array(0.0, f32)
    offs = []
    for e in range(TEN):
        o0 = (i0 == e).astype(f32)
        o1 = (i1 == e).astype(f32)
        p0 = jax.lax.dot_general(o0, mlane, (((1,), (0,)), ((), ())),
                                 preferred_element_type=f32)
        p1 = jax.lax.dot_general(o1, mlane, (((1,), (0,)), ((), ())),
                                 preferred_element_type=f32)
        rt0 = jnp.sum(o0, axis=1, keepdims=True)
        rt1 = jnp.sum(o1, axis=1, keepdims=True)
        pre0 = jax.lax.dot_general(mrow, rt0, (((1,), (0,)), ((), ())),
                                   preferred_element_type=f32)
        pre1 = jax.lax.dot_general(mrow, rt1, (((1,), (0,)), ((), ())),
                                   preferred_element_type=f32)
        c0 = jnp.sum(rt0)
        cnt = c0 + jnp.sum(rt1)
        pos0 = pos0 + o0 * (p0 + pre0 + off)
        pos1 = pos1 + o1 * (p1 + pre1 + c0 + off)
        offs.append(off)
        off = off + jnp.ceil(cnt / T_ROW) * T_ROW
    pos0_ref[...] = pos0.astype(I32)
    pos1_ref[...] = pos1.astype(I32)
    m = jax.lax.broadcasted_iota(I32, (1, 128), 1).astype(f32) * T_ROW
    te = sum(((m >= o).astype(I32)) for o in offs) - 1
    teid_ref[...] = te


def _k3b(i0, i1):
    return pl.pallas_call(
        _k3b_body,
        grid=(1,),
        in_specs=[
            pl.BlockSpec((16, 128), lambda i: (0, 0)),
            pl.BlockSpec((16, 128), lambda i: (0, 0)),
        ],
        out_specs=[
            pl.BlockSpec((16, 128), lambda i: (0, 0)),
            pl.BlockSpec((16, 128), lambda i: (0, 0)),
            pl.BlockSpec((1, 128), lambda i: (0, 0)),
        ],
        out_shape=[
            jax.ShapeDtypeStruct((16, 128), I32),
            jax.ShapeDtypeStruct((16, 128), I32),
            jax.ShapeDtypeStruct((1, 128), I32),
        ],
    )(i0, i1)


@functools.lru_cache(maxsize=None)
def _sc_kernels():
    mesh = plsc.VectorSubcoreMesh(core_axis_name="c", subcore_axis_name="s")

    @functools.partial(
        pl.kernel,
        out_type=jax.ShapeDtypeStruct((NP, DIM), f32),
        mesh=mesh,
        scratch_types=[
            pltpu.VMEM((_TPW,), I32),
            pltpu.VMEM((_TPW,), I32),
            pltpu.VMEM((_TPW, DIM), f32),
            pltpu.SemaphoreType.DMA,
        ],
    )
    def _sc_dispatch(p0_hbm, p1_hbm, x_hbm, out_hbm, i0_v, i1_v, val_v, sem):
        wid = lax.axis_index("s") * 2 + lax.axis_index("c")
        base = wid * _TPW
        pltpu.sync_copy(p0_hbm.at[wid], i0_v)
        pltpu.sync_copy(p1_hbm.at[wid], i1_v)
        pltpu.sync_copy(x_hbm.at[pl.ds(base, _TPW)], val_v)
        pltpu.async_copy(val_v, out_hbm.at[i0_v], sem).wait()
        pltpu.async_copy(val_v, out_hbm.at[i1_v], sem).wait()

    @functools.partial(
        pl.kernel,
        out_type=[
            jax.ShapeDtypeStruct((S, DIM), f32),
            jax.ShapeDtypeStruct((S, DIM), f32),
        ],
        mesh=mesh,
        scratch_types=[
            pltpu.VMEM((_TPW,), I32),
            pltpu.VMEM((_TPW, DIM), f32),
            pltpu.SemaphoreType.DMA,
        ],
    )
    def _sc_combine(p0_hbm, p1_hbm, o_hbm, y0_hbm, y1_hbm, i_v, rows_v, sem):
        wid = lax.axis_index("s") * 2 + lax.axis_index("c")
        base = wid * _TPW
        for src, dst in ((p0_hbm, y0_hbm), (p1_hbm, y1_hbm)):
            pltpu.sync_copy(src.at[wid], i_v)
            pltpu.async_copy(o_hbm.at[i_v], rows_v, sem).wait()
            pltpu.sync_copy(rows_v, dst.at[pl.ds(base, _TPW)])

    return _sc_dispatch, _sc_combine


_TPW = S // NW


def _gemm_body(teid_ref, xs_ref, w0_ref, w1_ref, w2_ref, o_ref):
    xs = xs_ref[...].astype(bf16)
    g = jnp.dot(xs, w0_ref[0].astype(bf16), preferred_element_type=f32)
    u = jnp.dot(xs, w1_ref[0].astype(bf16), preferred_element_type=f32)
    h = (jax.nn.silu(g) * u).astype(bf16)
    o = jax.lax.dot_general(h, w2_ref[0].astype(bf16), (((1,), (1,)), ((), ())),
                            preferred_element_type=f32)
    o_ref[...] = o


def _gemm(teid, xs, w0, w1, w2):
    grid_spec = pltpu.PrefetchScalarGridSpec(
        num_scalar_prefetch=1,
        grid=(NT,),
        in_specs=[
            pl.BlockSpec((T_ROW, DIM), lambda m, t: (m, 0)),
            pl.BlockSpec((1, DIM, EDIM), lambda m, t: (t[m], 0, 0)),
            pl.BlockSpec((1, DIM, EDIM), lambda m, t: (t[m], 0, 0)),
            pl.BlockSpec((1, DIM, EDIM), lambda m, t: (t[m], 0, 0)),
        ],
        out_specs=pl.BlockSpec((T_ROW, DIM), lambda m, t: (m, 0)),
    )
    return pl.pallas_call(
        _gemm_body,
        grid_spec=grid_spec,
        out_shape=jax.ShapeDtypeStruct((NP, DIM), f32),
    )(teid, xs, w0, w1, w2)


def _k5a_body(x_ref, up_ref, down_ref, out_ref):
    xu = jnp.dot(x_ref[...], up_ref[...], preferred_element_type=f32)
    x1, x2 = xu[:, :SDIM], xu[:, SDIM:]
    h = (jax.nn.silu(x1) * x2).astype(bf16)
    out_ref[...] = jnp.dot(h, down_ref[...], preferred_element_type=f32)


def _k5a(xffn, up_wb, down_wb):
    return pl.pallas_call(
        _k5a_body,
        grid=(S // SB,),
        in_specs=[
            pl.BlockSpec((SB, DIM), lambda i: (i, 0)),
            pl.BlockSpec((DIM, 2 * SDIM), lambda i: (0, 0)),
            pl.BlockSpec((SDIM, DIM), lambda i: (0, 0)),
        ],
        out_specs=pl.BlockSpec((SB, DIM), lambda i: (i, 0)),
        out_shape=jax.ShapeDtypeStruct((S, DIM), f32),
    )(xffn, up_wb, down_wb)


def _k5b_body(ys_ref, y0_ref, y1_ref, sc_ref, resid_ref, out_ref):
    s0 = sc_ref[:, 0:1]
    s1 = sc_ref[:, 1:2]
    y_moe = y0_ref[...] * s0 + y1_ref[...] * s1
    out_ref[...] = ys_ref[...] + y_moe + resid_ref[...]


def _k5b(ys, y0, y1, scores, resid):
    return pl.pallas_call(
        _k5b_body,
        grid=(S // SB,),
        in_specs=[
            pl.BlockSpec((SB, DIM), lambda i: (i, 0)),
            pl.BlockSpec((SB, DIM), lambda i: (i, 0)),
            pl.BlockSpec((SB, DIM), lambda i: (i, 0)),
            pl.BlockSpec((SB, 128), lambda i: (i, 0)),
            pl.BlockSpec((SB, DIM), lambda i: (i, 0)),
        ],
        out_specs=pl.BlockSpec((SB, DIM), lambda i: (i, 0)),
        out_shape=jax.ShapeDtypeStruct((S, DIM), f32),
    )(ys, y0, y1, scores, resid)


def kernel(x_input, indices, values, attn_norm_w, qkv_w, o_w, ffn_norm_w,
           keys_w, experts_w, up_w, down_w):
    x = x_input.reshape(S, DIM)
    o_wb = o_w.astype(bf16)
    up_wb = up_w.astype(bf16)
    down_wb = down_w.astype(bf16)
    keys_pad = jnp.pad(keys_w, ((0, 0), (0, 128 - TEN)))
    idx_pad = jnp.pad(indices.astype(jnp.int32), ((0, 0), (0, 128 - TOPK)),
                      constant_values=TEN)
    val_pad = jnp.pad(values, ((0, 0), (0, 128 - TOPK)), constant_values=NEG)

    xn, ta, tb = _k0(x, attn_norm_w.reshape(1, DIM))
    qkv = _k1(xn, qkv_w, ta, tb)
    attn = _k2(qkv)
    resid, xffn, xffn32, scores = _k3(attn, o_wb, x, ffn_norm_w.reshape(1, DIM),
                                      keys_pad, idx_pad, val_pad)
    i0 = indices[:, 0].astype(I32).reshape(16, 128)
    i1 = indices[:, 1].astype(I32).reshape(16, 128)
    pos0m, pos1m, teid_pad = _k3b(i0, i1)
    pos0 = pos0m.reshape(NW, _TPW)
    pos1 = pos1m.reshape(NW, _TPW)
    teid = teid_pad.reshape(128)[:NT]
    sc_dispatch, sc_combine = _sc_kernels()
    y_shared = _k5a(xffn, up_wb, down_wb)
    xs = sc_dispatch(pos0, pos1, xffn32)
    o_s = _gemm(teid, xs, experts_w[0], experts_w[1], experts_w[2])
    y0, y1 = sc_combine(pos0, pos1, o_s)
    out = _k5b(y_shared, y0, y1, scores, resid)
    return out.reshape(B, S, DIM)
